# baseline (speedup 1.0000x reference)
import functools

import jax
import jax.numpy as jnp
from jax import lax
from jax.experimental import pallas as pl
from jax.experimental.pallas import tpu as pltpu
from jax.experimental.pallas import tpu_sc as plsc

DIM = 32
B = 16384
L = 20
NEG = 5

NC = 2
NS = 16
NW = NC * NS
BPW = B // NW
CB = 64
NCHUNK = BPW // CB
NGRP = CB // 16

CTX_SL = CB * L // 128
PAD = DIM + 1


def _sc_body(ctx_i, cen, ctab, out_w,
             ci_v, ctx_st, cen_st, obp, wT, sem):
    cid = lax.axis_index("c")
    sid = lax.axis_index("s")
    wid = sid * NC + cid
    iota16 = lax.iota(jnp.int32, 16)

    pltpu.sync_copy(ctx_i.at[pl.ds(wid * (BPW * L // 128), BPW * L // 128)], ci_v)

    def chunk(k, carry):
        base = wid * BPW + k * CB
        cps = [pltpu.async_copy(
            ctab.at[ci_v.at[k * CTX_SL + j]],
            ctx_st.at[pl.ds(j * 128, 128)], sem) for j in range(CTX_SL)]
        cps.append(pltpu.async_copy(cen.at[pl.ds(base, CB)], cen_st, sem))
        for cp in cps:
            cp.wait()

        def bbody(b, carry2):
            bsp = jnp.full((16,), 1, jnp.int32) * b
            c0 = plsc.load_gather(cen_st, [bsp, iota16 * 2])
            c1 = plsc.load_gather(cen_st, [bsp, iota16 * 2 + 1])
            acc0 = jnp.zeros((16,), jnp.float32)
            acc1 = jnp.zeros((16,), jnp.float32)
            wsum = jnp.zeros((16,), jnp.float32)
            row0 = b * L
            for l in range(L):
                xr = ctx_st[row0 + l, :]
                x0, x1 = plsc.unpack(xr, format=plsc.PackFormat.INTERLEAVED,
                                     preferred_element_type=jnp.float32)
                d0 = x0 - c0
                d1 = x1 - c1
                s2 = d0 * d0 + d1 * d1
                tot = jnp.sum(s2)
                w = jnp.exp(jnp.broadcast_to(tot * -0.5, (16,)))
                acc0 = acc0 + w * x0
                acc1 = acc1 + w * x1
                wsum = wsum + w
            inv = 1.0 / (wsum + 1e-8)
            obp[b, pl.ds(0, 16)] = acc0 * inv
            obp[b, pl.ds(16, 16)] = acc1 * inv
            return carry2

        lax.fori_loop(0, CB, bbody, 0)

        def tbody(c, carry2):
            d = lax.rem(c, 16) * 2 + lax.div(c, 16)
            dsp = jnp.full((16,), 1, jnp.int32) * c
            for g in range(NGRP):
                bv = iota16 + g * 16
                wT[d, pl.ds(g * 16, 16)] = plsc.load_gather(obp, [bv, dsp])
            return carry2

        lax.fori_loop(0, DIM, tbody, 0)
        pltpu.sync_copy(wT, out_w.at[:, pl.ds(base, CB)])
        return carry

    lax.fori_loop(0, NCHUNK, chunk, 0)


@jax.jit
def _run(ctx_i, cen, ctab):
    mesh = plsc.VectorSubcoreMesh(core_axis_name="c", subcore_axis_name="s")
    f = pl.kernel(
        _sc_body,
        out_type=jax.ShapeDtypeStruct((DIM, B), jnp.float32),
        mesh=mesh,
        scratch_types=[
            pltpu.VMEM((BPW * L // 128, 128), jnp.int32),
            pltpu.VMEM((CB * L, DIM), jnp.bfloat16),
            pltpu.VMEM((CB, DIM), jnp.float32),
            pltpu.VMEM((CB, PAD), jnp.float32),
            pltpu.VMEM((DIM, CB), jnp.float32),
            pltpu.SemaphoreType.DMA,
        ],
        compiler_params=pltpu.CompilerParams(
            needs_layout_passes=False, use_tc_tiling_on_sc=False),
    )
    return f(ctx_i, cen, ctab)


def kernel(context, center, neg_samples, center_table, context_table):
    cen_vec = jnp.take(context_table, center, axis=0)
    neg_vecs = jnp.stack(
        [jnp.take(context_table, neg_samples[:, j], axis=0)
         for j in range(NEG)], axis=1)
    ctx_i = context.astype(jnp.int32).reshape(B * L // 128, 128)
    out_w = _run(ctx_i, cen_vec, center_table.astype(jnp.bfloat16))
    return (out_w.T, cen_vec, neg_vecs)


# speedup vs baseline: 1.3026x; 1.3026x over previous
import functools

import jax
import jax.numpy as jnp
from jax import lax
from jax.experimental import pallas as pl
from jax.experimental.pallas import tpu as pltpu
from jax.experimental.pallas import tpu_sc as plsc

DIM = 32
B = 16384
L = 20
NEG = 5

NC = 2
NS = 16
NW = NC * NS
BPW = B // NW
CB = 64
NCHUNK = BPW // CB
NGRP = CB // 16

CTX_SL = CB * L // 128
PAD = DIM + 1


def _sc_body(ctx_i, cen, ctab, out_w,
             ci_v, ctx_a, ctx_b, cen_a, cen_b, obp, wT, sem_a, sem_b):
    cid = lax.axis_index("c")
    sid = lax.axis_index("s")
    wid = sid * NC + cid
    iota16 = lax.iota(jnp.int32, 16)

    pltpu.sync_copy(ctx_i.at[pl.ds(wid * (BPW * L // 128), BPW * L // 128)], ci_v)

    def fire(k, ctx_st, cen_st, sem):
        base = wid * BPW + k * CB
        for j in range(CTX_SL):
            pltpu.async_copy(ctab.at[ci_v.at[k * CTX_SL + j]],
                             ctx_st.at[pl.ds(j * 128, 128)], sem)
        pltpu.async_copy(cen.at[pl.ds(base, CB)], cen_st, sem)

    def drain(ctx_st, cen_st, sem):
        pltpu.make_async_copy(ctab.at[pl.ds(0, CB * L)], ctx_st, sem).wait()
        pltpu.make_async_copy(cen.at[pl.ds(0, CB)], cen_st, sem).wait()

    def process(k, ctx_st, cen_st):
        def bbody(b, carry2):
            c0 = cen_st[b, pl.ds(0, 16)]
            c1 = cen_st[b, pl.ds(16, 16)]
            acc0 = jnp.zeros((16,), jnp.float32)
            acc1 = jnp.zeros((16,), jnp.float32)
            wsum = jnp.zeros((16,), jnp.float32)
            row0 = b * L
            for l in range(L):
                x0 = ctx_st[row0 + l, pl.ds(0, 16)]
                x1 = ctx_st[row0 + l, pl.ds(16, 16)]
                d0 = x0 - c0
                d1 = x1 - c1
                s2 = d0 * d0 + d1 * d1
                tot = jnp.sum(s2)
                w = jnp.exp(jnp.broadcast_to(tot * -0.5, (16,)))
                acc0 = acc0 + w * x0
                acc1 = acc1 + w * x1
                wsum = wsum + w
            inv = 1.0 / (wsum + 1e-8)
            obp[b, pl.ds(0, 16)] = acc0 * inv
            obp[b, pl.ds(16, 16)] = acc1 * inv
            return carry2

        lax.fori_loop(0, CB, bbody, 0)

        def tbody(d, carry2):
            dsp = jnp.full((16,), 1, jnp.int32) * d
            for g in range(NGRP):
                bv = iota16 + g * 16
                wT[d, pl.ds(g * 16, 16)] = plsc.load_gather(obp, [bv, dsp])
            return carry2

        lax.fori_loop(0, DIM, tbody, 0)
        pltpu.sync_copy(wT, out_w.at[:, pl.ds(wid * BPW + k * CB, CB)])

    fire(0, ctx_a, cen_a, sem_a)

    def pair(k2, carry):
        k = k2 * 2
        fire(k + 1, ctx_b, cen_b, sem_b)
        drain(ctx_a, cen_a, sem_a)
        process(k, ctx_a, cen_a)

        @pl.when(k2 < NCHUNK // 2 - 1)
        def _():
            fire(k + 2, ctx_a, cen_a, sem_a)

        drain(ctx_b, cen_b, sem_b)
        process(k + 1, ctx_b, cen_b)
        return carry

    lax.fori_loop(0, NCHUNK // 2, pair, 0)


@jax.jit
def _run(ctx_i, cen, ctab):
    mesh = plsc.VectorSubcoreMesh(core_axis_name="c", subcore_axis_name="s")
    f = pl.kernel(
        _sc_body,
        out_type=jax.ShapeDtypeStruct((DIM, B), jnp.float32),
        mesh=mesh,
        scratch_types=[
            pltpu.VMEM((BPW * L // 128, 128), jnp.int32),
            pltpu.VMEM((CB * L, DIM), jnp.float32),
            pltpu.VMEM((CB * L, DIM), jnp.float32),
            pltpu.VMEM((CB, DIM), jnp.float32),
            pltpu.VMEM((CB, DIM), jnp.float32),
            pltpu.VMEM((CB, PAD), jnp.float32),
            pltpu.VMEM((DIM, CB), jnp.float32),
            pltpu.SemaphoreType.DMA,
            pltpu.SemaphoreType.DMA,
        ],
        compiler_params=pltpu.CompilerParams(
            needs_layout_passes=False, use_tc_tiling_on_sc=False),
    )
    return f(ctx_i, cen, ctab)


def kernel(context, center, neg_samples, center_table, context_table):
    cen_vec = jnp.take(context_table, center, axis=0)
    neg_vecs = jnp.stack(
        [jnp.take(context_table, neg_samples[:, j], axis=0)
         for j in range(NEG)], axis=1)
    ctx_i = context.astype(jnp.int32).reshape(B * L // 128, 128)
    out_w = _run(ctx_i, cen_vec, center_table)
    return (out_w.T, cen_vec, neg_vecs)


# speedup vs baseline: 1.3038x; 1.0010x over previous
import jax
import jax.numpy as jnp
from jax import lax
from jax.experimental import pallas as pl
from jax.experimental.pallas import tpu as pltpu
from jax.experimental.pallas import tpu_sc as plsc

DIM = 32
B = 16384
L = 20
NEG = 5

NC = 2
NS = 16
NW = NC * NS
BPW = B // NW
CB = 64
NCHUNK = BPW // CB
NGRP = CB // 16

CTX_SL = CB * L // 128
PAD = DIM + 1


def _sc_body(ctx_i, cen, ctab, out_w,
             ci_v, ctx_a, ctx_b, cen_a, cen_b, obp, wT, sem_a, sem_b):
    cid = lax.axis_index("c")
    sid = lax.axis_index("s")
    wid = sid * NC + cid
    iota16 = lax.iota(jnp.int32, 16)

    pltpu.sync_copy(ctx_i.at[pl.ds(wid * (BPW * L // 128), BPW * L // 128)], ci_v)

    def fire(k, ctx_st, cen_st, sem):
        base = wid * BPW + k * CB
        for j in range(CTX_SL):
            pltpu.async_copy(ctab.at[ci_v.at[k * CTX_SL + j]],
                             ctx_st.at[pl.ds(j * 128, 128)], sem)
        pltpu.async_copy(cen.at[pl.ds(base, CB)], cen_st, sem)

    def drain(ctx_st, cen_st, sem):
        pltpu.make_async_copy(ctab.at[pl.ds(0, CB * L)], ctx_st, sem).wait()
        pltpu.make_async_copy(cen.at[pl.ds(0, CB)], cen_st, sem).wait()

    def process(k, ctx_st, cen_st):
        def bbody(b, carry2):
            c0 = cen_st[b, pl.ds(0, 16)]
            c1 = cen_st[b, pl.ds(16, 16)]
            acc0 = jnp.zeros((16,), jnp.float32)
            acc1 = jnp.zeros((16,), jnp.float32)
            wsum = jnp.zeros((16,), jnp.float32)
            row0 = b * L
            for l in range(L):
                x0 = ctx_st[row0 + l, pl.ds(0, 16)]
                x1 = ctx_st[row0 + l, pl.ds(16, 16)]
                d0 = x0 - c0
                d1 = x1 - c1
                s2 = d0 * d0 + d1 * d1
                tot = jnp.sum(s2)
                w = jnp.exp(jnp.broadcast_to(tot * -0.5, (16,)))
                acc0 = acc0 + w * x0
                acc1 = acc1 + w * x1
                wsum = wsum + w
            inv = 1.0 / (wsum + 1e-8)
            obp[b, pl.ds(0, 16)] = acc0 * inv
            obp[b, pl.ds(16, 16)] = acc1 * inv
            return carry2

        lax.fori_loop(0, CB, bbody, 0)

        def tbody(d, carry2):
            dsp = jnp.full((16,), 1, jnp.int32) * d
            for g in range(NGRP):
                bv = iota16 + g * 16
                wT[d, pl.ds(g * 16, 16)] = plsc.load_gather(obp, [bv, dsp])
            return carry2

        lax.fori_loop(0, DIM, tbody, 0)
        pltpu.sync_copy(wT, out_w.at[:, pl.ds(wid * BPW + k * CB, CB)])

    fire(0, ctx_a, cen_a, sem_a)

    def pair(k2, carry):
        k = k2 * 2
        fire(k + 1, ctx_b, cen_b, sem_b)
        drain(ctx_a, cen_a, sem_a)
        process(k, ctx_a, cen_a)

        @pl.when(k2 < NCHUNK // 2 - 1)
        def _():
            fire(k + 2, ctx_a, cen_a, sem_a)

        drain(ctx_b, cen_b, sem_b)
        process(k + 1, ctx_b, cen_b)
        return carry

    lax.fori_loop(0, NCHUNK // 2, pair, 0)


@jax.jit
def _run(ctx_i, cen, ctab):
    mesh = plsc.VectorSubcoreMesh(core_axis_name="c", subcore_axis_name="s")
    f = pl.kernel(
        _sc_body,
        out_type=jax.ShapeDtypeStruct((DIM, B), jnp.float32),
        mesh=mesh,
        scratch_types=[
            pltpu.VMEM((BPW * L // 128, 128), jnp.int32),
            pltpu.VMEM((CB * L, DIM), jnp.float32),
            pltpu.VMEM((CB * L, DIM), jnp.float32),
            pltpu.VMEM((CB, DIM), jnp.float32),
            pltpu.VMEM((CB, DIM), jnp.float32),
            pltpu.VMEM((CB, PAD), jnp.float32),
            pltpu.VMEM((DIM, CB), jnp.float32),
            pltpu.SemaphoreType.DMA,
            pltpu.SemaphoreType.DMA,
        ],
        compiler_params=pltpu.CompilerParams(
            needs_layout_passes=False, use_tc_tiling_on_sc=False),
    )
    return f(ctx_i, cen, ctab)


def kernel(context, center, neg_samples, center_table, context_table):
    cen_vec = jnp.take(context_table, center, axis=0)
    neg_vecs = jnp.stack(
        [jnp.take(context_table, neg_samples[:, j], axis=0)
         for j in range(NEG)], axis=1)
    ctx_i = context.astype(jnp.int32).reshape(B * L // 128, 128)
    out_w = _run(ctx_i, cen_vec, center_table)
    return (out_w.T, cen_vec, neg_vecs)
